# Initial kernel scaffold; baseline (speedup 1.0000x reference)
import jax
import jax.numpy as jnp
from jax.experimental import pallas as pl


def kernel(x, edge_index, W, att_src, att_dst, bias):
    raise NotImplementedError("write your pallas kernel here")


# speedup vs baseline: 27.9771x; 27.9771x over previous
import jax
import jax.numpy as jnp
from jax import lax
from jax.experimental import pallas as pl
from jax.experimental.pallas import tpu as pltpu
from jax.experimental.pallas import tpu_sc as plsc

NEG_SLOPE = 0.2
NC, NS, LANES = 2, 16, 16
NW = NC * NS
CHUNK = 128
NODE_PAD = 128


def _tc_project(x_pad, W, att2):
    NP, D = x_pad.shape

    def body(x_ref, w_ref, a_ref, xp_ref, a2_ref):
        xp = jnp.dot(x_ref[...], w_ref[...], preferred_element_type=jnp.float32)
        xp_ref[...] = xp
        a2_ref[...] = lax.dot_general(
            a_ref[...], xp, (((1,), (1,)), ((), ())),
            preferred_element_type=jnp.float32)

    return pl.pallas_call(
        body,
        out_shape=[
            jax.ShapeDtypeStruct((NP, D), jnp.float32),
            jax.ShapeDtypeStruct((2, NP), jnp.float32),
        ],
    )(x_pad, W, att2)


def _sc_aggregate(xp, a2, src3, dst3, n_edges_real):
    NP, D = xp.shape
    CHUNKS = src3.shape[1]
    rpt = NP // NS

    def body(xp_hbm, a2_hbm, src_hbm, dst_hbm, out_hbm, den_hbm,
             a_src_v, a_dst_v, sidx_v, didx_v, alpha_v, rows_v, den_v,
             accum_sh, den_sh, sem):
        cid = lax.axis_index("c")
        sid = lax.axis_index("s")
        wid = cid * NS + sid

        pltpu.sync_copy(a2_hbm.at[0], a_src_v)
        pltpu.sync_copy(a2_hbm.at[1], a_dst_v)

        zeros16 = jnp.zeros((LANES,), jnp.float32)

        def zrow(r, _):
            for c in range(D // LANES):
                rows_v[r, pl.ds(c * LANES, LANES)] = zeros16
            return 0

        lax.fori_loop(0, CHUNK, zrow, 0)
        for i in range(CHUNK // LANES):
            alpha_v[pl.ds(i * LANES, LANES)] = zeros16
        for off in range(0, rpt, CHUNK):
            cnt = min(CHUNK, rpt - off)
            pltpu.sync_copy(rows_v.at[pl.ds(0, cnt)],
                            accum_sh.at[pl.ds(sid * rpt + off, cnt)])
            pltpu.sync_copy(alpha_v.at[pl.ds(0, cnt)],
                            den_sh.at[pl.ds(sid * rpt + off, cnt)])
        plsc.subcore_barrier()

        iota16 = lax.iota(jnp.int32, LANES)

        def chunk_body(j, _):
            pltpu.sync_copy(src_hbm.at[wid, j], sidx_v)
            pltpu.sync_copy(dst_hbm.at[wid, j], didx_v)
            cp = pltpu.async_copy(xp_hbm.at[sidx_v], rows_v, sem)
            base = (wid * CHUNKS + j) * CHUNK
            for i in range(CHUNK // LANES):
                sidx = sidx_v[pl.ds(i * LANES, LANES)]
                didx = didx_v[pl.ds(i * LANES, LANES)]
                al = (plsc.load_gather(a_src_v, [sidx])
                      + plsc.load_gather(a_dst_v, [didx]))
                al = jnp.where(al > 0, al, NEG_SLOPE * al)
                ev = jnp.exp(al)
                eid = base + i * LANES + iota16
                ev = jnp.where(eid < n_edges_real, ev, 0.0)
                alpha_v[pl.ds(i * LANES, LANES)] = ev
            cp.wait()

            def srow(r, _):
                asp = plsc.load_gather(
                    alpha_v, [jnp.broadcast_to(r, (LANES,)).astype(jnp.int32)])
                for c in range(D // LANES):
                    rows_v[r, pl.ds(c * LANES, LANES)] = (
                        rows_v[r, pl.ds(c * LANES, LANES)] * asp)
                return 0

            lax.fori_loop(0, CHUNK, srow, 0)
            pltpu.sync_copy(rows_v, accum_sh.at[didx_v], add=True)
            pltpu.sync_copy(alpha_v, den_sh.at[didx_v], add=True)
            return 0

        lax.fori_loop(0, CHUNKS, chunk_body, 0)
        plsc.subcore_barrier()

        for off in range(0, rpt, CHUNK):
            cnt = min(CHUNK, rpt - off)
            pltpu.sync_copy(accum_sh.at[pl.ds(sid * rpt + off, cnt)],
                            rows_v.at[pl.ds(0, cnt)])
            pltpu.sync_copy(rows_v.at[pl.ds(0, cnt)],
                            out_hbm.at[cid, pl.ds(sid * rpt + off, cnt)])
        pltpu.sync_copy(den_sh.at[pl.ds(sid * rpt, rpt)], den_v)
        pltpu.sync_copy(den_v, den_hbm.at[pl.ds(cid * NP + sid * rpt, rpt)])

    mesh = plsc.VectorSubcoreMesh(core_axis_name="c", subcore_axis_name="s")
    return pl.kernel(
        body,
        out_type=[
            jax.ShapeDtypeStruct((NC, NP, D), jnp.float32),
            jax.ShapeDtypeStruct((NC * NP,), jnp.float32),
        ],
        mesh=mesh,
        compiler_params=pltpu.CompilerParams(needs_layout_passes=False),
        scratch_types=[
            pltpu.VMEM((NP,), jnp.float32),
            pltpu.VMEM((NP,), jnp.float32),
            pltpu.VMEM((CHUNK,), jnp.int32),
            pltpu.VMEM((CHUNK,), jnp.int32),
            pltpu.VMEM((CHUNK,), jnp.float32),
            pltpu.VMEM((CHUNK, D), jnp.float32),
            pltpu.VMEM((rpt,), jnp.float32),
            pltpu.VMEM_SHARED((NP, D), jnp.float32),
            pltpu.VMEM_SHARED((NP,), jnp.float32),
            pltpu.SemaphoreType.DMA,
        ],
    )(xp, a2, src3, dst3)


def _tc_combine(accd, dend3, bias2, n_real):
    D = bias2.shape[1]

    def body(acc_ref, den_ref, b_ref, o_ref):
        num = acc_ref[0] + acc_ref[1]
        den = den_ref[0] + den_ref[1]
        o_ref[...] = (num[:n_real] / (den[:n_real] + 1e-16)) + b_ref[...]

    return pl.pallas_call(
        body,
        out_shape=jax.ShapeDtypeStruct((n_real, D), jnp.float32),
    )(accd, dend3, bias2)


def kernel(x, edge_index, W, att_src, att_dst, bias):
    n, d_in = x.shape
    e = edge_index.shape[1]

    npad = -n % NODE_PAD
    NP = n + npad
    x_pad = jnp.concatenate([x, jnp.zeros((npad, d_in), x.dtype)], axis=0)
    att2 = jnp.stack([att_src, att_dst])

    xp, a2 = _tc_project(x_pad, W, att2)

    ee = e + n
    epad = -ee % (NW * CHUNK)
    chunks = (ee + epad) // (NW * CHUNK)
    loop = jnp.arange(n, dtype=jnp.int32)
    fill = jnp.arange(epad, dtype=jnp.int32) % NP
    src3 = jnp.concatenate([edge_index[0], loop, fill]).reshape(NW, chunks, CHUNK)
    dst3 = jnp.concatenate([edge_index[1], loop, fill]).reshape(NW, chunks, CHUNK)

    accd, dend = _sc_aggregate(xp, a2, src3, dst3, ee)
    dend3 = dend.reshape(NC, NP, 1)
    return _tc_combine(accd, dend3, bias[None, :], n)


# speedup vs baseline: 48.1333x; 1.7204x over previous
import jax
import jax.numpy as jnp
from jax import lax
from jax.experimental import pallas as pl
from jax.experimental.pallas import tpu as pltpu
from jax.experimental.pallas import tpu_sc as plsc

NEG_SLOPE = 0.2
NC, NS, LANES = 2, 16, 16
NW = NC * NS
CHUNK = 128
NODE_PAD = 128


def _tc_project(x_pad, W, att2):
    NP, D = x_pad.shape

    def body(x_ref, w_ref, a_ref, xp_ref, a2_ref):
        xp = jnp.dot(x_ref[...], w_ref[...], preferred_element_type=jnp.float32)
        xp_ref[...] = xp
        a2_ref[...] = lax.dot_general(
            a_ref[...], xp, (((1,), (1,)), ((), ())),
            preferred_element_type=jnp.float32)

    return pl.pallas_call(
        body,
        out_shape=[
            jax.ShapeDtypeStruct((NP, D), jnp.float32),
            jax.ShapeDtypeStruct((2, NP), jnp.float32),
        ],
    )(x_pad, W, att2)


def _sc_aggregate(xp, a_srcH, a_dstH, src3, dst3, n_edges_real):
    NP, D = xp.shape
    CHUNKS = src3.shape[1]
    rpt = NP // NS

    def body(xp_hbm, as_hbm, ad_hbm, src_hbm, dst_hbm, out_hbm, den_hbm,
             sidx_v, didx_v, asv_v, adv_v, alpha_v, rows_v, den_v,
             accum_sh, den_sh, sem_i, sem_l, sem_g, sem_sr, sem_sd):
        cid = lax.axis_index("c")
        sid = lax.axis_index("s")
        wid = cid * NS + sid

        zeros16 = jnp.zeros((LANES,), jnp.float32)

        def zrow(r, _):
            for c in range(D // LANES):
                rows_v[0, r, pl.ds(c * LANES, LANES)] = zeros16
            return 0

        lax.fori_loop(0, CHUNK, zrow, 0)
        for i in range(CHUNK // LANES):
            alpha_v[0, pl.ds(i * LANES, LANES)] = zeros16
        for off in range(0, rpt, CHUNK):
            cnt = min(CHUNK, rpt - off)
            pltpu.sync_copy(rows_v.at[0, pl.ds(0, cnt)],
                            accum_sh.at[pl.ds(sid * rpt + off, cnt)])
            pltpu.sync_copy(alpha_v.at[0, pl.ds(0, cnt)],
                            den_sh.at[pl.ds(sid * rpt + off, cnt)])
        plsc.subcore_barrier()

        iota16 = lax.iota(jnp.int32, LANES)

        def issue_idx(j, slot):
            pltpu.async_copy(src_hbm.at[wid, j], sidx_v.at[slot], sem_i.at[slot])
            pltpu.async_copy(dst_hbm.at[wid, j], didx_v.at[slot], sem_i.at[slot])

        def wait_idx(slot):
            pltpu.make_async_copy(src_hbm.at[wid, 0], sidx_v.at[slot],
                                  sem_i.at[slot]).wait()
            pltpu.make_async_copy(dst_hbm.at[wid, 0], didx_v.at[slot],
                                  sem_i.at[slot]).wait()

        def issue_logit(islot, slot):
            pltpu.async_copy(as_hbm.at[sidx_v.at[islot]], asv_v.at[slot],
                             sem_l.at[slot])
            pltpu.async_copy(ad_hbm.at[didx_v.at[islot]], adv_v.at[slot],
                             sem_l.at[slot])

        def wait_logit(slot):
            pltpu.make_async_copy(as_hbm.at[pl.ds(0, CHUNK)], asv_v.at[slot],
                                  sem_l.at[slot]).wait()
            pltpu.make_async_copy(ad_hbm.at[pl.ds(0, CHUNK)], adv_v.at[slot],
                                  sem_l.at[slot]).wait()

        def issue_rows(islot, slot):
            pltpu.async_copy(xp_hbm.at[sidx_v.at[islot]], rows_v.at[slot],
                             sem_g.at[slot])

        def wait_rows(slot):
            pltpu.make_async_copy(xp_hbm.at[pl.ds(0, CHUNK)], rows_v.at[slot],
                                  sem_g.at[slot]).wait()

        def issue_scatter(islot, slot):
            pltpu.async_copy(rows_v.at[slot], accum_sh.at[didx_v.at[islot]],
                             sem_sr.at[slot], add=True)
            pltpu.async_copy(alpha_v.at[slot], den_sh.at[didx_v.at[islot]],
                             sem_sd.at[slot], add=True)

        def wait_scatter(slot):
            pltpu.make_async_copy(xp_hbm.at[pl.ds(0, CHUNK)], rows_v.at[slot],
                                  sem_sr.at[slot]).wait()
            pltpu.make_async_copy(as_hbm.at[pl.ds(0, CHUNK)], alpha_v.at[slot],
                                  sem_sd.at[slot]).wait()

        pltpu.sync_copy(src_hbm.at[wid, 0], sidx_v.at[0])
        pltpu.sync_copy(dst_hbm.at[wid, 0], didx_v.at[0])
        issue_idx(1, 1)
        issue_logit(0, 0)
        issue_rows(0, 0)

        def chunk_body(j, _):
            b = lax.rem(j, 2)
            nb = 1 - b
            i3 = lax.rem(j, 3)
            i3n = lax.rem(j + 1, 3)
            i3nn = lax.rem(j + 2, 3)

            @pl.when(j >= 1)
            def _():
                wait_scatter(nb)

            @pl.when(j <= CHUNKS - 2)
            def _():
                wait_idx(i3n)
                issue_logit(i3n, nb)
                issue_rows(i3n, nb)

            wait_rows(b)
            wait_logit(b)

            @pl.when(j <= CHUNKS - 3)
            def _():
                issue_idx(j + 2, i3nn)

            base = (wid * CHUNKS + j) * CHUNK
            for i in range(CHUNK // LANES):
                al = (asv_v[b, pl.ds(i * LANES, LANES)]
                      + adv_v[b, pl.ds(i * LANES, LANES)])
                al = jnp.where(al > 0, al, NEG_SLOPE * al)
                ev = jnp.exp(al)
                eid = base + i * LANES + iota16
                ev = jnp.where(eid < n_edges_real, ev, 0.0)
                alpha_v[b, pl.ds(i * LANES, LANES)] = ev

            def srow(r, _):
                asp = plsc.load_gather(
                    alpha_v,
                    [jnp.broadcast_to(b, (LANES,)).astype(jnp.int32),
                     jnp.broadcast_to(r, (LANES,)).astype(jnp.int32)])
                for c in range(D // LANES):
                    rows_v[b, r, pl.ds(c * LANES, LANES)] = (
                        rows_v[b, r, pl.ds(c * LANES, LANES)] * asp)
                return 0

            lax.fori_loop(0, CHUNK, srow, 0)
            issue_scatter(i3, b)
            return 0

        lax.fori_loop(0, CHUNKS, chunk_body, 0)
        wait_scatter((CHUNKS - 1) % 2)
        plsc.subcore_barrier()

        for off in range(0, rpt, CHUNK):
            cnt = min(CHUNK, rpt - off)
            pltpu.sync_copy(accum_sh.at[pl.ds(sid * rpt + off, cnt)],
                            rows_v.at[0, pl.ds(0, cnt)])
            pltpu.sync_copy(rows_v.at[0, pl.ds(0, cnt)],
                            out_hbm.at[cid, pl.ds(sid * rpt + off, cnt)])
        pltpu.sync_copy(den_sh.at[pl.ds(sid * rpt, rpt)], den_v)
        pltpu.sync_copy(den_v, den_hbm.at[pl.ds(cid * NP + sid * rpt, rpt)])

    mesh = plsc.VectorSubcoreMesh(core_axis_name="c", subcore_axis_name="s")
    return pl.kernel(
        body,
        out_type=[
            jax.ShapeDtypeStruct((NC, NP, D), jnp.float32),
            jax.ShapeDtypeStruct((NC * NP,), jnp.float32),
        ],
        mesh=mesh,
        compiler_params=pltpu.CompilerParams(needs_layout_passes=False),
        scratch_types=[
            pltpu.VMEM((3, CHUNK), jnp.int32),
            pltpu.VMEM((3, CHUNK), jnp.int32),
            pltpu.VMEM((2, CHUNK), jnp.float32),
            pltpu.VMEM((2, CHUNK), jnp.float32),
            pltpu.VMEM((2, CHUNK), jnp.float32),
            pltpu.VMEM((2, CHUNK, D), jnp.float32),
            pltpu.VMEM((rpt,), jnp.float32),
            pltpu.VMEM_SHARED((NP, D), jnp.float32),
            pltpu.VMEM_SHARED((NP,), jnp.float32),
            pltpu.SemaphoreType.DMA((3,)),
            pltpu.SemaphoreType.DMA((2,)),
            pltpu.SemaphoreType.DMA((2,)),
            pltpu.SemaphoreType.DMA((2,)),
            pltpu.SemaphoreType.DMA((2,)),
        ],
    )(xp, a_srcH, a_dstH, src3, dst3)


def _tc_combine(accd, dend3, bias2, n_real):
    D = bias2.shape[1]

    def body(acc_ref, den_ref, b_ref, o_ref):
        num = acc_ref[0] + acc_ref[1]
        den = den_ref[0] + den_ref[1]
        o_ref[...] = (num[:n_real] / (den[:n_real] + 1e-16)) + b_ref[...]

    return pl.pallas_call(
        body,
        out_shape=jax.ShapeDtypeStruct((n_real, D), jnp.float32),
    )(accd, dend3, bias2)


def kernel(x, edge_index, W, att_src, att_dst, bias):
    n, d_in = x.shape
    e = edge_index.shape[1]

    npad = -n % NODE_PAD
    NP = n + npad
    x_pad = jnp.concatenate([x, jnp.zeros((npad, d_in), x.dtype)], axis=0)
    att2 = jnp.stack([att_src, att_dst])

    xp, a2 = _tc_project(x_pad, W, att2)

    ee = e + n
    epad = -ee % (NW * CHUNK)
    chunks = (ee + epad) // (NW * CHUNK)
    loop = jnp.arange(n, dtype=jnp.int32)
    fill = jnp.arange(epad, dtype=jnp.int32) % NP
    src3 = jnp.concatenate([edge_index[0], loop, fill]).reshape(NW, chunks, CHUNK)
    dst3 = jnp.concatenate([edge_index[1], loop, fill]).reshape(NW, chunks, CHUNK)

    accd, dend = _sc_aggregate(xp, a2[0], a2[1], src3, dst3, ee)
    dend3 = dend.reshape(NC, NP, 1)
    return _tc_combine(accd, dend3, bias[None, :], n)
